# baseline (speedup 1.0000x reference)
import jax
import jax.numpy as jnp
from jax.experimental import pallas as pl

B = 16384
C = 128
BLOCK_ROWS = 2048


def _body(t_ref, w_ref, b_ref, out_ref):
    out_ref[...] = jnp.cos(t_ref[...] * w_ref[...] + b_ref[...])


def kernel(src, dst, time_diffs, W_lin, b_lin):
    del src, dst
    t = time_diffs.reshape(B, 1)
    w = W_lin.reshape(1, C)
    b = b_lin.reshape(1, C)
    grid = (B // BLOCK_ROWS,)
    return pl.pallas_call(
        _body,
        grid=grid,
        in_specs=[
            pl.BlockSpec((BLOCK_ROWS, 1), lambda i: (i, 0)),
            pl.BlockSpec((1, C), lambda i: (0, 0)),
            pl.BlockSpec((1, C), lambda i: (0, 0)),
        ],
        out_specs=pl.BlockSpec((BLOCK_ROWS, C), lambda i: (i, 0)),
        out_shape=jax.ShapeDtypeStruct((B, C), jnp.float32),
    )(t, w, b)


# speedup vs baseline: 2.0835x; 2.0835x over previous
import jax
import jax.numpy as jnp
from jax.experimental import pallas as pl

B = 16384
C = 128
BLOCK_ROWS = 2048

TWO_PI = 6.283185307179586
INV_2PI = 0.15915494309189535
C0 = 0.9999982491220226
C1 = -0.49999251123729715
C2 = 0.04165902522376602
C3 = -0.0013857590092085854
C4 = 2.419642869336448e-05
C5 = -2.1969776888275973e-07


def _cos_poly(x):
    r = x - jnp.round(x * INV_2PI) * TWO_PI
    u = r * r
    return ((((C5 * u + C4) * u + C3) * u + C2) * u + C1) * u + C0


def _body(t_ref, w_ref, b_ref, out_ref):
    out_ref[...] = _cos_poly(t_ref[...] * w_ref[...] + b_ref[...])


def kernel(src, dst, time_diffs, W_lin, b_lin):
    del src, dst
    t = time_diffs.reshape(B, 1)
    w = W_lin.reshape(1, C)
    b = b_lin.reshape(1, C)
    grid = (B // BLOCK_ROWS,)
    return pl.pallas_call(
        _body,
        grid=grid,
        in_specs=[
            pl.BlockSpec((BLOCK_ROWS, 1), lambda i: (i, 0)),
            pl.BlockSpec((1, C), lambda i: (0, 0)),
            pl.BlockSpec((1, C), lambda i: (0, 0)),
        ],
        out_specs=pl.BlockSpec((BLOCK_ROWS, C), lambda i: (i, 0)),
        out_shape=jax.ShapeDtypeStruct((B, C), jnp.float32),
    )(t, w, b)
